# Initial kernel scaffold; baseline (speedup 1.0000x reference)
import jax
import jax.numpy as jnp
from jax.experimental import pallas as pl


def kernel(X, A, W0a, a0a_s, a0a_n, W0b, a0b_s, a0b_n, pk, W1a, a1a_s, a1a_n, W1b, a1b_s, a1b_n, Wua, aua_s, aua_n, Wub, aub_s, aub_n, Wea, aea_s, aea_n, Web, aeb_s, aeb_n):
    raise NotImplementedError("write your pallas kernel here")


# speedup vs baseline: 1.2398x; 1.2398x over previous
import functools

import jax
import jax.numpy as jnp
from jax.experimental import pallas as pl
from jax.experimental.pallas import tpu as pltpu

B_, N_, F_ = 4, 1024, 128
H_, C_ = 4, 128
HC_ = H_ * C_
K_ = 512
NEG_ = -1e9


def _mask_body(a_ref, m_ref, *, bm, n):
    j = pl.program_id(1)
    a = a_ref[0]
    r = j * bm + jax.lax.broadcasted_iota(jnp.int32, (bm, n), 0)
    c = jax.lax.broadcasted_iota(jnp.int32, (bm, n), 1)
    m_ref[0] = ((a > 0) | (r == c)).astype(jnp.int8)


def _mask_call(a):
    b, n, _ = a.shape
    bm = 256
    return pl.pallas_call(
        functools.partial(_mask_body, bm=bm, n=n),
        grid=(b, n // bm),
        in_specs=[pl.BlockSpec((1, bm, n), lambda i, j: (i, j, 0))],
        out_specs=pl.BlockSpec((1, bm, n), lambda i, j: (i, j, 0)),
        out_shape=jax.ShapeDtypeStruct((b, n, n), jnp.int8),
    )(a)


def _proj_body(x_ref, w_ref, a_ref, xp_ref, st_ref):
    xp = jnp.dot(x_ref[0], w_ref[...], preferred_element_type=jnp.float32)
    xp_ref[0] = xp
    for h in range(H_):
        blk = xp[:, h * C_:(h + 1) * C_]
        st_ref[0, h, :] = jnp.sum(blk * a_ref[0, h * C_:(h + 1) * C_][None, :], axis=1)
        st_ref[0, H_ + h, :] = jnp.sum(blk * a_ref[1, h * C_:(h + 1) * C_][None, :], axis=1)


def _proj_call(x, w, avec):
    b, n, fin = x.shape
    bp = 256
    return pl.pallas_call(
        _proj_body,
        grid=(b, n // bp),
        in_specs=[
            pl.BlockSpec((1, bp, fin), lambda i, j: (i, j, 0)),
            pl.BlockSpec((fin, HC_), lambda i, j: (0, 0)),
            pl.BlockSpec((2, HC_), lambda i, j: (0, 0)),
        ],
        out_specs=[
            pl.BlockSpec((1, bp, HC_), lambda i, j: (i, j, 0)),
            pl.BlockSpec((1, 8, bp), lambda i, j: (i, 0, j)),
        ],
        out_shape=[
            jax.ShapeDtypeStruct((b, n, HC_), jnp.float32),
            jax.ShapeDtypeStruct((b, 8, n), jnp.float32),
        ],
    )(x, w, avec)


def _proj2_body(x1_ref, x2_ref, w1_ref, w2_ref, a_ref, xp_ref, st_ref):
    xp = jnp.dot(x1_ref[0], w1_ref[...], preferred_element_type=jnp.float32)
    xp = xp + jnp.dot(x2_ref[0], w2_ref[...], preferred_element_type=jnp.float32)
    xp_ref[0] = xp
    for h in range(H_):
        blk = xp[:, h * C_:(h + 1) * C_]
        st_ref[0, h, :] = jnp.sum(blk * a_ref[0, h * C_:(h + 1) * C_][None, :], axis=1)
        st_ref[0, H_ + h, :] = jnp.sum(blk * a_ref[1, h * C_:(h + 1) * C_][None, :], axis=1)


def _proj2_call(x1, x2, w1, w2, avec):
    b, n, f1 = x1.shape
    f2 = x2.shape[-1]
    bp = 256
    return pl.pallas_call(
        _proj2_body,
        grid=(b, n // bp),
        in_specs=[
            pl.BlockSpec((1, bp, f1), lambda i, j: (i, j, 0)),
            pl.BlockSpec((1, bp, f2), lambda i, j: (i, j, 0)),
            pl.BlockSpec((f1, HC_), lambda i, j: (0, 0)),
            pl.BlockSpec((f2, HC_), lambda i, j: (0, 0)),
            pl.BlockSpec((2, HC_), lambda i, j: (0, 0)),
        ],
        out_specs=[
            pl.BlockSpec((1, bp, HC_), lambda i, j: (i, j, 0)),
            pl.BlockSpec((1, 8, bp), lambda i, j: (i, 0, j)),
        ],
        out_shape=[
            jax.ShapeDtypeStruct((b, n, HC_), jnp.float32),
            jax.ShapeDtypeStruct((b, 8, n), jnp.float32),
        ],
    )(x1, x2, w1, w2, avec)


def _attn_heads(mask_ref, st_ref, xp_ref, *, bn, n):
    j = pl.program_id(1)
    mask = mask_ref[0] != 0
    for h in range(H_):
        s = st_ref[0, h, pl.ds(j * bn, bn)]
        t = st_ref[0, H_ + h, :]
        l = s[:, None] + t[None, :]
        l = jnp.where(l >= 0, l, 0.2 * l)
        l = jnp.where(mask, l, NEG_)
        m = jnp.max(l, axis=1, keepdims=True)
        p = jnp.exp(l - m)
        attn = p / jnp.sum(p, axis=1, keepdims=True)
        yield jnp.dot(attn, xp_ref[0, :, h * C_:(h + 1) * C_],
                      preferred_element_type=jnp.float32)


def _attn_concat_body(mask_ref, st_ref, xp_ref, o_ref, *, bn, n):
    for h, oh in enumerate(_attn_heads(mask_ref, st_ref, xp_ref, bn=bn, n=n)):
        o_ref[0, :, h * C_:(h + 1) * C_] = oh


def _attn_mean(mask_ref, st_ref, xp_ref, *, bn, n):
    acc = None
    for oh in _attn_heads(mask_ref, st_ref, xp_ref, bn=bn, n=n):
        acc = oh if acc is None else acc + oh
    return jnp.maximum(acc * (1.0 / H_), 0.0)


def _attn_mean_body(mask_ref, st_ref, xp_ref, o_ref, *, bn, n):
    o_ref[0] = _attn_mean(mask_ref, st_ref, xp_ref, bn=bn, n=n)


def _attn_mean_add_body(mask_ref, st_ref, xp_ref, d_ref, o_ref, *, bn, n):
    o_ref[0] = _attn_mean(mask_ref, st_ref, xp_ref, bn=bn, n=n) + d_ref[0]


def _attn_mean_pool_body(mask_ref, st_ref, xp_ref, pk_ref, o_ref, g_ref, y_ref,
                         *, bn, n):
    j = pl.program_id(1)
    out = _attn_mean(mask_ref, st_ref, xp_ref, bn=bn, n=n)
    o_ref[0] = out
    pk = pk_ref[0]
    kn = pk / (jnp.sqrt(jnp.sum(pk * pk)) + 1e-12)
    y = jnp.sum(out * kn[None, :], axis=1)
    y_ref[0, 0, pl.ds(j * bn, bn)] = y
    g_ref[0] = out * jnp.tanh(y)[:, None]


def _attn_call(xp, st, mask8, mode, down=None, pk=None):
    b, n, _ = xp.shape
    bn = 256
    grid = (b, n // bn)
    in_specs = [
        pl.BlockSpec((1, bn, n), lambda i, j: (i, j, 0)),
        pl.BlockSpec((1, 8, n), lambda i, j: (i, 0, 0)),
        pl.BlockSpec((1, n, HC_), lambda i, j: (i, 0, 0)),
    ]
    args = [mask8, st, xp]
    if mode == "concat":
        body = functools.partial(_attn_concat_body, bn=bn, n=n)
        out_specs = pl.BlockSpec((1, bn, HC_), lambda i, j: (i, j, 0))
        out_shape = jax.ShapeDtypeStruct((b, n, HC_), jnp.float32)
    elif mode == "mean":
        body = functools.partial(_attn_mean_body, bn=bn, n=n)
        out_specs = pl.BlockSpec((1, bn, C_), lambda i, j: (i, j, 0))
        out_shape = jax.ShapeDtypeStruct((b, n, C_), jnp.float32)
    elif mode == "mean_add":
        body = functools.partial(_attn_mean_add_body, bn=bn, n=n)
        in_specs.append(pl.BlockSpec((1, bn, C_), lambda i, j: (i, j, 0)))
        args.append(down)
        out_specs = pl.BlockSpec((1, bn, C_), lambda i, j: (i, j, 0))
        out_shape = jax.ShapeDtypeStruct((b, n, C_), jnp.float32)
    elif mode == "mean_pool":
        body = functools.partial(_attn_mean_pool_body, bn=bn, n=n)
        in_specs.append(pl.BlockSpec((1, C_), lambda i, j: (0, 0)))
        args.append(pk)
        out_specs = [
            pl.BlockSpec((1, bn, C_), lambda i, j: (i, j, 0)),
            pl.BlockSpec((1, bn, C_), lambda i, j: (i, j, 0)),
            pl.BlockSpec((1, 1, n), lambda i, j: (i, 0, 0)),
        ]
        out_shape = [
            jax.ShapeDtypeStruct((b, n, C_), jnp.float32),
            jax.ShapeDtypeStruct((b, n, C_), jnp.float32),
            jax.ShapeDtypeStruct((b, 1, n), jnp.float32),
        ]
    return pl.pallas_call(
        body, grid=grid, in_specs=in_specs, out_specs=out_specs,
        out_shape=out_shape,
    )(*args)


def kernel(X, A, W0a, a0a_s, a0a_n, W0b, a0b_s, a0b_n, pk,
           W1a, a1a_s, a1a_n, W1b, a1b_s, a1b_n,
           Wua, aua_s, aua_n, Wub, aub_s, aub_n,
           Wea, aea_s, aea_n, Web, aeb_s, aeb_n):
    def wf(w):
        return w.reshape(w.shape[0], HC_)

    def av(a_s, a_n):
        return jnp.stack([a_s.reshape(HC_), a_n.reshape(HC_)])

    mask8 = _mask_call(A)

    xp, st = _proj_call(X, wf(W0a), av(a0a_s, a0a_n))
    h0a = _attn_call(xp, st, mask8, "concat")
    xp, st = _proj_call(h0a, wf(W0b), av(a0b_s, a0b_n))
    down, hg, y = _attn_call(xp, st, mask8, "mean_pool", pk=pk.reshape(1, F_))

    _, idx = jax.lax.top_k(y[:, 0, :], K_)

    hp = jnp.take_along_axis(hg, idx[:, :, None], axis=1)
    Ap = jnp.take_along_axis(A, idx[:, :, None], axis=1)
    Ap = jnp.take_along_axis(Ap, idx[:, None, :], axis=2)
    mp8 = _mask_call(Ap)

    xp, st = _proj_call(hp, wf(W1a), av(a1a_s, a1a_n))
    h1a = _attn_call(xp, st, mp8, "concat")
    xp, st = _proj_call(h1a, wf(W1b), av(a1b_s, a1b_n))
    h1b = _attn_call(xp, st, mp8, "mean")

    bidx = jnp.arange(B_)[:, None]
    hu0 = jnp.zeros((B_, N_, C_), jnp.float32).at[bidx, idx].set(h1b)

    xp, st = _proj_call(hu0, wf(Wua), av(aua_s, aua_n))
    hua = _attn_call(xp, st, mask8, "concat")
    xp, st = _proj_call(hua, wf(Wub), av(aub_s, aub_n))
    hu = _attn_call(xp, st, mask8, "mean_add", down=down)

    xp, st = _proj2_call(hu, X, wf(Wea[:C_]), wf(Wea[C_:]), av(aea_s, aea_n))
    hea = _attn_call(xp, st, mask8, "concat")
    xp, st = _proj_call(hea, wf(Web), av(aeb_s, aeb_n))
    out = _attn_call(xp, st, mask8, "mean")
    return out


# speedup vs baseline: 1.3620x; 1.0986x over previous
import functools

import jax
import jax.numpy as jnp
from jax.experimental import pallas as pl
from jax.experimental.pallas import tpu as pltpu

B_, N_, F_ = 4, 1024, 128
H_, C_ = 4, 128
HC_ = H_ * C_
K_ = 512
NEG_ = -1e9


def _stj(astack, xp):
    return jax.lax.dot_general(astack, xp, (((0,), (1,)), ((), ())),
                               preferred_element_type=jnp.float32)


def _mask_body(a_ref, m_ref, *, bm, n):
    j = pl.program_id(1)
    a = a_ref[0]
    r = j * bm + jax.lax.broadcasted_iota(jnp.int32, (bm, n), 0)
    c = jax.lax.broadcasted_iota(jnp.int32, (bm, n), 1)
    m_ref[0] = ((a > 0) | (r == c)).astype(jnp.int8)


def _mask_call(a):
    b, n, _ = a.shape
    bm = 256
    return pl.pallas_call(
        functools.partial(_mask_body, bm=bm, n=n),
        grid=(b, n // bm),
        in_specs=[pl.BlockSpec((1, bm, n), lambda i, j: (i, j, 0))],
        out_specs=pl.BlockSpec((1, bm, n), lambda i, j: (i, j, 0)),
        out_shape=jax.ShapeDtypeStruct((b, n, n), jnp.int8),
    )(a)


def _proj_body(x_ref, w_ref, a_ref, xp_ref, st_ref):
    xp = jnp.dot(x_ref[0], w_ref[...], preferred_element_type=jnp.float32)
    xp_ref[0] = xp
    st_ref[0] = _stj(a_ref[...], xp)


def _proj_call(x, w, astack):
    b, n, fin = x.shape
    bp = 256
    return pl.pallas_call(
        _proj_body,
        grid=(b, n // bp),
        in_specs=[
            pl.BlockSpec((1, bp, fin), lambda i, j: (i, j, 0)),
            pl.BlockSpec((fin, HC_), lambda i, j: (0, 0)),
            pl.BlockSpec((HC_, 8), lambda i, j: (0, 0)),
        ],
        out_specs=[
            pl.BlockSpec((1, bp, HC_), lambda i, j: (i, j, 0)),
            pl.BlockSpec((1, 8, bp), lambda i, j: (i, 0, j)),
        ],
        out_shape=[
            jax.ShapeDtypeStruct((b, n, HC_), jnp.float32),
            jax.ShapeDtypeStruct((b, 8, n), jnp.float32),
        ],
    )(x, w, astack)


def _attn_heads(mask_ref, st_ref, xp_ref, *, bn, n):
    j = pl.program_id(1)
    mask = mask_ref[0] != 0
    outs = []
    for h in range(H_):
        s = st_ref[0, h, pl.ds(j * bn, bn)]
        t = st_ref[0, H_ + h, :]
        l = s[:, None] + t[None, :]
        l = jnp.where(mask, jnp.maximum(l, 0.2 * l), NEG_)
        m = jnp.max(l, axis=1, keepdims=True)
        p = jnp.exp(l - m)
        inv = 1.0 / jnp.sum(p, axis=1, keepdims=True)
        oh = jnp.dot(p, xp_ref[0, :, h * C_:(h + 1) * C_],
                     preferred_element_type=jnp.float32)
        outs.append(oh * inv)
    return outs


def _concat_out(mask_ref, st_ref, xp_ref, *, bn, n):
    return jnp.concatenate(_attn_heads(mask_ref, st_ref, xp_ref, bn=bn, n=n),
                           axis=1)


def _mean_out(mask_ref, st_ref, xp_ref, *, bn, n):
    outs = _attn_heads(mask_ref, st_ref, xp_ref, bn=bn, n=n)
    acc = outs[0] + outs[1] + outs[2] + outs[3]
    return jnp.maximum(acc * (1.0 / H_), 0.0)


def _attn_cat_proj_body(mask_ref, st_ref, xp_ref, w_ref, a_ref,
                        xpo_ref, sto_ref, *, bn, n):
    out = _concat_out(mask_ref, st_ref, xp_ref, bn=bn, n=n)
    xpo = jnp.dot(out, w_ref[...], preferred_element_type=jnp.float32)
    xpo_ref[0] = xpo
    sto_ref[0] = _stj(a_ref[...], xpo)


def _attn_mean_body(mask_ref, st_ref, xp_ref, o_ref, *, bn, n):
    o_ref[0] = _mean_out(mask_ref, st_ref, xp_ref, bn=bn, n=n)


def _attn_mean_pool_body(mask_ref, st_ref, xp_ref, pk_ref, o_ref, g_ref, y_ref,
                         *, bn, n):
    j = pl.program_id(1)
    out = _mean_out(mask_ref, st_ref, xp_ref, bn=bn, n=n)
    o_ref[0] = out
    pk = pk_ref[0]
    kn = pk / (jnp.sqrt(jnp.sum(pk * pk)) + 1e-12)
    y = jnp.sum(out * kn[None, :], axis=1)
    y_ref[0, 0, pl.ds(j * bn, bn)] = y
    g_ref[0] = out * jnp.tanh(y)[:, None]


def _attn_mean_add_proj2_body(mask_ref, st_ref, xp_ref, d_ref, x2_ref,
                              w1_ref, w2_ref, a_ref, xpo_ref, sto_ref,
                              *, bn, n):
    hu = _mean_out(mask_ref, st_ref, xp_ref, bn=bn, n=n) + d_ref[0]
    xpo = jnp.dot(hu, w1_ref[...], preferred_element_type=jnp.float32)
    xpo = xpo + jnp.dot(x2_ref[0], w2_ref[...],
                        preferred_element_type=jnp.float32)
    xpo_ref[0] = xpo
    sto_ref[0] = _stj(a_ref[...], xpo)


def _attn_call(xp, st, mask8, mode, **kw):
    b, n, _ = xp.shape
    bn = 256
    grid = (b, n // bn)
    in_specs = [
        pl.BlockSpec((1, bn, n), lambda i, j: (i, j, 0)),
        pl.BlockSpec((1, 8, n), lambda i, j: (i, 0, 0)),
        pl.BlockSpec((1, n, HC_), lambda i, j: (i, 0, 0)),
    ]
    args = [mask8, st, xp]
    xpo_spec = pl.BlockSpec((1, bn, HC_), lambda i, j: (i, j, 0))
    sto_spec = pl.BlockSpec((1, 8, bn), lambda i, j: (i, 0, j))
    xpo_shapes = [jax.ShapeDtypeStruct((b, n, HC_), jnp.float32),
                  jax.ShapeDtypeStruct((b, 8, n), jnp.float32)]
    if mode == "cat_proj":
        body = functools.partial(_attn_cat_proj_body, bn=bn, n=n)
        in_specs += [pl.BlockSpec((HC_, HC_), lambda i, j: (0, 0)),
                     pl.BlockSpec((HC_, 8), lambda i, j: (0, 0))]
        args += [kw["w"], kw["astack"]]
        out_specs = [xpo_spec, sto_spec]
        out_shape = xpo_shapes
    elif mode == "mean":
        body = functools.partial(_attn_mean_body, bn=bn, n=n)
        out_specs = pl.BlockSpec((1, bn, C_), lambda i, j: (i, j, 0))
        out_shape = jax.ShapeDtypeStruct((b, n, C_), jnp.float32)
    elif mode == "mean_pool":
        body = functools.partial(_attn_mean_pool_body, bn=bn, n=n)
        in_specs.append(pl.BlockSpec((1, C_), lambda i, j: (0, 0)))
        args.append(kw["pk"])
        out_specs = [
            pl.BlockSpec((1, bn, C_), lambda i, j: (i, j, 0)),
            pl.BlockSpec((1, bn, C_), lambda i, j: (i, j, 0)),
            pl.BlockSpec((1, 1, n), lambda i, j: (i, 0, 0)),
        ]
        out_shape = [
            jax.ShapeDtypeStruct((b, n, C_), jnp.float32),
            jax.ShapeDtypeStruct((b, n, C_), jnp.float32),
            jax.ShapeDtypeStruct((b, 1, n), jnp.float32),
        ]
    elif mode == "mean_add_proj2":
        body = functools.partial(_attn_mean_add_proj2_body, bn=bn, n=n)
        in_specs += [
            pl.BlockSpec((1, bn, C_), lambda i, j: (i, j, 0)),
            pl.BlockSpec((1, bn, F_), lambda i, j: (i, j, 0)),
            pl.BlockSpec((C_, HC_), lambda i, j: (0, 0)),
            pl.BlockSpec((F_, HC_), lambda i, j: (0, 0)),
            pl.BlockSpec((HC_, 8), lambda i, j: (0, 0)),
        ]
        args += [kw["down"], kw["x2"], kw["w1"], kw["w2"], kw["astack"]]
        out_specs = [xpo_spec, sto_spec]
        out_shape = xpo_shapes
    return pl.pallas_call(
        body, grid=grid, in_specs=in_specs, out_specs=out_specs,
        out_shape=out_shape,
    )(*args)


def kernel(X, A, W0a, a0a_s, a0a_n, W0b, a0b_s, a0b_n, pk,
           W1a, a1a_s, a1a_n, W1b, a1b_s, a1b_n,
           Wua, aua_s, aua_n, Wub, aub_s, aub_n,
           Wea, aea_s, aea_n, Web, aeb_s, aeb_n):
    def wf(w):
        return w.reshape(w.shape[0], HC_)

    def av(a_s, a_n):
        z = jnp.zeros((HC_, 2 * H_), jnp.float32)
        for h in range(H_):
            z = z.at[h * C_:(h + 1) * C_, h].set(a_s[h])
            z = z.at[h * C_:(h + 1) * C_, H_ + h].set(a_n[h])
        return z

    mask8 = _mask_call(A)

    xp, st = _proj_call(X, wf(W0a), av(a0a_s, a0a_n))
    xp, st = _attn_call(xp, st, mask8, "cat_proj",
                        w=wf(W0b), astack=av(a0b_s, a0b_n))
    down, hg, y = _attn_call(xp, st, mask8, "mean_pool", pk=pk.reshape(1, F_))

    _, idx = jax.lax.top_k(y[:, 0, :], K_)

    hp = jnp.take_along_axis(hg, idx[:, :, None], axis=1)
    Ap = jnp.take_along_axis(A, idx[:, :, None], axis=1)
    Ap = jnp.take_along_axis(Ap, idx[:, None, :], axis=2)
    mp8 = _mask_call(Ap)

    xp, st = _proj_call(hp, wf(W1a), av(a1a_s, a1a_n))
    xp, st = _attn_call(xp, st, mp8, "cat_proj",
                        w=wf(W1b), astack=av(a1b_s, a1b_n))
    h1b = _attn_call(xp, st, mp8, "mean")

    bidx = jnp.arange(B_)[:, None]
    hu0 = jnp.zeros((B_, N_, C_), jnp.float32).at[bidx, idx].set(h1b)

    xp, st = _proj_call(hu0, wf(Wua), av(aua_s, aua_n))
    xp, st = _attn_call(xp, st, mask8, "cat_proj",
                        w=wf(Wub), astack=av(aub_s, aub_n))
    xp, st = _attn_call(xp, st, mask8, "mean_add_proj2", down=down, x2=X,
                        w1=wf(Wea[:C_]), w2=wf(Wea[C_:]),
                        astack=av(aea_s, aea_n))
    xp, st = _attn_call(xp, st, mask8, "cat_proj",
                        w=wf(Web), astack=av(aeb_s, aeb_n))
    out = _attn_call(xp, st, mask8, "mean")
    return out


# speedup vs baseline: 1.5591x; 1.1447x over previous
import functools

import jax
import jax.numpy as jnp
from jax.experimental import pallas as pl
from jax.experimental.pallas import tpu as pltpu

B_, N_, F_ = 4, 1024, 128
H_, C_ = 4, 128
HC_ = H_ * C_
K_ = 512


def _stj(astack, xp):
    return jax.lax.dot_general(astack, xp, (((0,), (1,)), ((), ())),
                               preferred_element_type=jnp.float32)


def _leaky(x):
    return jnp.maximum(x, 0.2 * x)


def _mask_from(a_ref, j, bm, n):
    a = a_ref[0]
    r = j * bm + jax.lax.broadcasted_iota(jnp.int32, (bm, n), 0)
    c = jax.lax.broadcasted_iota(jnp.int32, (bm, n), 1)
    return ((a > 0) | (r == c)).astype(jnp.int8)


def _proj_body(x_ref, w_ref, a_ref, xp_ref, st_ref):
    xp = jnp.dot(x_ref[0], w_ref[...], preferred_element_type=jnp.float32)
    xp_ref[0] = xp
    st_ref[0] = _stj(a_ref[...], xp)


def _proj_mask_body(x_ref, adj_ref, w_ref, a_ref, xp_ref, st_ref, m_ref,
                    *, bp, n):
    _proj_body(x_ref, w_ref, a_ref, xp_ref, st_ref)
    m_ref[0] = _mask_from(adj_ref, pl.program_id(1), bp, n)


def _proj_call(x, w, astack, adj=None):
    b, n, fin = x.shape
    bp = 256
    in_specs = [
        pl.BlockSpec((1, bp, fin), lambda i, j: (i, j, 0)),
        pl.BlockSpec((fin, HC_), lambda i, j: (0, 0)),
        pl.BlockSpec((HC_, 8), lambda i, j: (0, 0)),
    ]
    out_specs = [
        pl.BlockSpec((1, bp, HC_), lambda i, j: (i, j, 0)),
        pl.BlockSpec((1, 8, bp), lambda i, j: (i, 0, j)),
    ]
    out_shape = [
        jax.ShapeDtypeStruct((b, n, HC_), jnp.float32),
        jax.ShapeDtypeStruct((b, 8, n), jnp.float32),
    ]
    if adj is None:
        body = _proj_body
        args = (x, w, astack)
    else:
        body = functools.partial(_proj_mask_body, bp=bp, n=n)
        in_specs.insert(1, pl.BlockSpec((1, bp, n), lambda i, j: (i, j, 0)))
        out_specs.append(pl.BlockSpec((1, bp, n), lambda i, j: (i, j, 0)))
        out_shape.append(jax.ShapeDtypeStruct((b, n, n), jnp.int8))
        args = (x, adj, w, astack)
    return pl.pallas_call(
        body, grid=(b, n // bp), in_specs=in_specs, out_specs=out_specs,
        out_shape=out_shape,
    )(*args)


def _attn_heads(mask_ref, st_ref, xp_ref, *, bn, n):
    j = pl.program_id(1)
    maskf = mask_ref[0].astype(jnp.float32)
    outs = []
    for h in range(H_):
        s = st_ref[0, h, pl.ds(j * bn, bn)]
        t = st_ref[0, H_ + h, :]
        mhat = _leaky(s + jnp.max(t))[:, None]
        p = jnp.exp(_leaky(s[:, None] + t[None, :]) - mhat) * maskf
        inv = 1.0 / jnp.sum(p, axis=1, keepdims=True)
        oh = jnp.dot(p, xp_ref[0, :, h * C_:(h + 1) * C_],
                     preferred_element_type=jnp.float32)
        outs.append(oh * inv)
    return outs


def _concat_out(mask_ref, st_ref, xp_ref, *, bn, n):
    return jnp.concatenate(_attn_heads(mask_ref, st_ref, xp_ref, bn=bn, n=n),
                           axis=1)


def _mean_out(mask_ref, st_ref, xp_ref, *, bn, n):
    outs = _attn_heads(mask_ref, st_ref, xp_ref, bn=bn, n=n)
    acc = outs[0] + outs[1] + outs[2] + outs[3]
    return jnp.maximum(acc * (1.0 / H_), 0.0)


def _attn_cat_proj_body(mask_ref, st_ref, xp_ref, w_ref, a_ref,
                        xpo_ref, sto_ref, *, bn, n):
    out = _concat_out(mask_ref, st_ref, xp_ref, bn=bn, n=n)
    xpo = jnp.dot(out, w_ref[...], preferred_element_type=jnp.float32)
    xpo_ref[0] = xpo
    sto_ref[0] = _stj(a_ref[...], xpo)


def _attn_mean_body(mask_ref, st_ref, xp_ref, o_ref, *, bn, n):
    o_ref[0] = _mean_out(mask_ref, st_ref, xp_ref, bn=bn, n=n)


def _attn_mean_pool_body(mask_ref, st_ref, xp_ref, pk_ref, o_ref, g_ref, y_ref,
                         *, bn, n):
    j = pl.program_id(1)
    out = _mean_out(mask_ref, st_ref, xp_ref, bn=bn, n=n)
    o_ref[0] = out
    pk = pk_ref[0]
    kn = pk / (jnp.sqrt(jnp.sum(pk * pk)) + 1e-12)
    y = jnp.sum(out * kn[None, :], axis=1)
    y_ref[0, 0, pl.ds(j * bn, bn)] = y
    g_ref[0] = out * jnp.tanh(y)[:, None]


def _attn_mean_add_proj2_body(mask_ref, st_ref, xp_ref, d_ref, x2_ref,
                              w1_ref, w2_ref, a_ref, xpo_ref, sto_ref,
                              *, bn, n):
    hu = _mean_out(mask_ref, st_ref, xp_ref, bn=bn, n=n) + d_ref[0]
    xpo = jnp.dot(hu, w1_ref[...], preferred_element_type=jnp.float32)
    xpo = xpo + jnp.dot(x2_ref[0], w2_ref[...],
                        preferred_element_type=jnp.float32)
    xpo_ref[0] = xpo
    sto_ref[0] = _stj(a_ref[...], xpo)


def _attn_call(xp, st, mask8, mode, **kw):
    b, n, _ = xp.shape
    bn = 512
    grid = (b, n // bn)
    in_specs = [
        pl.BlockSpec((1, bn, n), lambda i, j: (i, j, 0)),
        pl.BlockSpec((1, 8, n), lambda i, j: (i, 0, 0)),
        pl.BlockSpec((1, n, HC_), lambda i, j: (i, 0, 0)),
    ]
    args = [mask8, st, xp]
    xpo_spec = pl.BlockSpec((1, bn, HC_), lambda i, j: (i, j, 0))
    sto_spec = pl.BlockSpec((1, 8, bn), lambda i, j: (i, 0, j))
    xpo_shapes = [jax.ShapeDtypeStruct((b, n, HC_), jnp.float32),
                  jax.ShapeDtypeStruct((b, 8, n), jnp.float32)]
    if mode == "cat_proj":
        body = functools.partial(_attn_cat_proj_body, bn=bn, n=n)
        in_specs += [pl.BlockSpec((HC_, HC_), lambda i, j: (0, 0)),
                     pl.BlockSpec((HC_, 8), lambda i, j: (0, 0))]
        args += [kw["w"], kw["astack"]]
        out_specs = [xpo_spec, sto_spec]
        out_shape = xpo_shapes
    elif mode == "mean":
        body = functools.partial(_attn_mean_body, bn=bn, n=n)
        out_specs = pl.BlockSpec((1, bn, C_), lambda i, j: (i, j, 0))
        out_shape = jax.ShapeDtypeStruct((b, n, C_), jnp.float32)
    elif mode == "mean_pool":
        body = functools.partial(_attn_mean_pool_body, bn=bn, n=n)
        in_specs.append(pl.BlockSpec((1, C_), lambda i, j: (0, 0)))
        args.append(kw["pk"])
        out_specs = [
            pl.BlockSpec((1, bn, C_), lambda i, j: (i, j, 0)),
            pl.BlockSpec((1, bn, C_), lambda i, j: (i, j, 0)),
            pl.BlockSpec((1, 1, n), lambda i, j: (i, 0, 0)),
        ]
        out_shape = [
            jax.ShapeDtypeStruct((b, n, C_), jnp.float32),
            jax.ShapeDtypeStruct((b, n, C_), jnp.float32),
            jax.ShapeDtypeStruct((b, 1, n), jnp.float32),
        ]
    elif mode == "mean_add_proj2":
        body = functools.partial(_attn_mean_add_proj2_body, bn=bn, n=n)
        in_specs += [
            pl.BlockSpec((1, bn, C_), lambda i, j: (i, j, 0)),
            pl.BlockSpec((1, bn, F_), lambda i, j: (i, j, 0)),
            pl.BlockSpec((C_, HC_), lambda i, j: (0, 0)),
            pl.BlockSpec((F_, HC_), lambda i, j: (0, 0)),
            pl.BlockSpec((HC_, 8), lambda i, j: (0, 0)),
        ]
        args += [kw["down"], kw["x2"], kw["w1"], kw["w2"], kw["astack"]]
        out_specs = [xpo_spec, sto_spec]
        out_shape = xpo_shapes
    return pl.pallas_call(
        body, grid=grid, in_specs=in_specs, out_specs=out_specs,
        out_shape=out_shape,
    )(*args)


def kernel(X, A, W0a, a0a_s, a0a_n, W0b, a0b_s, a0b_n, pk,
           W1a, a1a_s, a1a_n, W1b, a1b_s, a1b_n,
           Wua, aua_s, aua_n, Wub, aub_s, aub_n,
           Wea, aea_s, aea_n, Web, aeb_s, aeb_n):
    def wf(w):
        return w.reshape(w.shape[0], HC_)

    def av(a_s, a_n):
        z = jnp.zeros((HC_, 2 * H_), jnp.float32)
        for h in range(H_):
            z = z.at[h * C_:(h + 1) * C_, h].set(a_s[h])
            z = z.at[h * C_:(h + 1) * C_, H_ + h].set(a_n[h])
        return z

    xp, st, mask8 = _proj_call(X, wf(W0a), av(a0a_s, a0a_n), adj=A)
    xp, st = _attn_call(xp, st, mask8, "cat_proj",
                        w=wf(W0b), astack=av(a0b_s, a0b_n))
    down, hg, y = _attn_call(xp, st, mask8, "mean_pool", pk=pk.reshape(1, F_))

    _, idx = jax.lax.top_k(y[:, 0, :], K_)

    hp = jnp.take_along_axis(hg, idx[:, :, None], axis=1)
    Ap = jnp.take_along_axis(A, idx[:, :, None], axis=1)
    Ap = jnp.take_along_axis(Ap, idx[:, None, :], axis=2)

    xp, st, mp8 = _proj_call(hp, wf(W1a), av(a1a_s, a1a_n), adj=Ap)
    xp, st = _attn_call(xp, st, mp8, "cat_proj",
                        w=wf(W1b), astack=av(a1b_s, a1b_n))
    h1b = _attn_call(xp, st, mp8, "mean")

    bidx = jnp.arange(B_)[:, None]
    hu0 = jnp.zeros((B_, N_, C_), jnp.float32).at[bidx, idx].set(h1b)

    xp, st = _proj_call(hu0, wf(Wua), av(aua_s, aua_n))
    xp, st = _attn_call(xp, st, mask8, "cat_proj",
                        w=wf(Wub), astack=av(aub_s, aub_n))
    xp, st = _attn_call(xp, st, mask8, "mean_add_proj2", down=down, x2=X,
                        w1=wf(Wea[:C_]), w2=wf(Wea[C_:]),
                        astack=av(aea_s, aea_n))
    xp, st = _attn_call(xp, st, mask8, "cat_proj",
                        w=wf(Web), astack=av(aeb_s, aeb_n))
    out = _attn_call(xp, st, mask8, "mean")
    return out


# speedup vs baseline: 1.7412x; 1.1168x over previous
import functools

import jax
import jax.numpy as jnp
from jax import lax
from jax.experimental import pallas as pl
from jax.experimental.pallas import tpu as pltpu
from jax.experimental.pallas import tpu_sc as plsc

B_, N_, F_ = 4, 1024, 128
H_, C_ = 4, 128
HC_ = H_ * C_
K_ = 512


def _stj(astack, xp):
    return jax.lax.dot_general(astack, xp, (((0,), (1,)), ((), ())),
                               preferred_element_type=jnp.float32)


def _leaky(x):
    return jnp.maximum(x, 0.2 * x)


def _mask_from(a_ref, j, bm, n):
    a = a_ref[0]
    r = j * bm + jax.lax.broadcasted_iota(jnp.int32, (bm, n), 0)
    c = jax.lax.broadcasted_iota(jnp.int32, (bm, n), 1)
    return ((a > 0) | (r == c)).astype(jnp.int8)


def _proj_body(x_ref, w_ref, a_ref, xp_ref, st_ref):
    xp = jnp.dot(x_ref[0], w_ref[...], preferred_element_type=jnp.float32)
    xp_ref[0] = xp
    st_ref[0] = _stj(a_ref[...], xp)


def _proj_mask_body(x_ref, adj_ref, w_ref, a_ref, xp_ref, st_ref, m_ref,
                    *, bp, n):
    _proj_body(x_ref, w_ref, a_ref, xp_ref, st_ref)
    m_ref[0] = _mask_from(adj_ref, pl.program_id(1), bp, n)


def _proj_colsel_body(x_ref, ar_ref, oh_ref, w_ref, a_ref,
                      xp_ref, st_ref, m_ref, *, bp, n):
    _proj_body(x_ref, w_ref, a_ref, xp_ref, st_ref)
    ap = jnp.dot(ar_ref[0], oh_ref[0], preferred_element_type=jnp.float32)
    j = pl.program_id(1)
    r = j * bp + jax.lax.broadcasted_iota(jnp.int32, (bp, n), 0)
    c = jax.lax.broadcasted_iota(jnp.int32, (bp, n), 1)
    m_ref[0] = ((ap > 0) | (r == c)).astype(jnp.int8)


def _proj_colsel_call(x, ar, oh, w, astack):
    b, n, fin = x.shape
    bp = 256
    return pl.pallas_call(
        functools.partial(_proj_colsel_body, bp=bp, n=n),
        grid=(b, n // bp),
        in_specs=[
            pl.BlockSpec((1, bp, fin), lambda i, j: (i, j, 0)),
            pl.BlockSpec((1, bp, N_), lambda i, j: (i, j, 0)),
            pl.BlockSpec((1, N_, n), lambda i, j: (i, 0, 0)),
            pl.BlockSpec((fin, HC_), lambda i, j: (0, 0)),
            pl.BlockSpec((HC_, 8), lambda i, j: (0, 0)),
        ],
        out_specs=[
            pl.BlockSpec((1, bp, HC_), lambda i, j: (i, j, 0)),
            pl.BlockSpec((1, 8, bp), lambda i, j: (i, 0, j)),
            pl.BlockSpec((1, bp, n), lambda i, j: (i, j, 0)),
        ],
        out_shape=[
            jax.ShapeDtypeStruct((b, n, HC_), jnp.float32),
            jax.ShapeDtypeStruct((b, 8, n), jnp.float32),
            jax.ShapeDtypeStruct((b, n, n), jnp.int8),
        ],
    )(x, ar, oh, w, astack)


def _proj_call(x, w, astack, adj=None):
    b, n, fin = x.shape
    bp = 256
    in_specs = [
        pl.BlockSpec((1, bp, fin), lambda i, j: (i, j, 0)),
        pl.BlockSpec((fin, HC_), lambda i, j: (0, 0)),
        pl.BlockSpec((HC_, 8), lambda i, j: (0, 0)),
    ]
    out_specs = [
        pl.BlockSpec((1, bp, HC_), lambda i, j: (i, j, 0)),
        pl.BlockSpec((1, 8, bp), lambda i, j: (i, 0, j)),
    ]
    out_shape = [
        jax.ShapeDtypeStruct((b, n, HC_), jnp.float32),
        jax.ShapeDtypeStruct((b, 8, n), jnp.float32),
    ]
    if adj is None:
        body = _proj_body
        args = (x, w, astack)
    else:
        body = functools.partial(_proj_mask_body, bp=bp, n=n)
        in_specs.insert(1, pl.BlockSpec((1, bp, n), lambda i, j: (i, j, 0)))
        out_specs.append(pl.BlockSpec((1, bp, n), lambda i, j: (i, j, 0)))
        out_shape.append(jax.ShapeDtypeStruct((b, n, n), jnp.int8))
        args = (x, adj, w, astack)
    return pl.pallas_call(
        body, grid=(b, n // bp), in_specs=in_specs, out_specs=out_specs,
        out_shape=out_shape,
    )(*args)


def _attn_heads(mask_ref, st_ref, xp_ref, *, bn, n):
    j = pl.program_id(1)
    maskf = mask_ref[0].astype(jnp.float32)
    outs = []
    for h in range(H_):
        s = st_ref[0, h, pl.ds(j * bn, bn)]
        t = st_ref[0, H_ + h, :]
        mhat = _leaky(s + jnp.max(t))[:, None]
        p = jnp.exp(_leaky(s[:, None] + t[None, :]) - mhat) * maskf
        inv = 1.0 / jnp.sum(p, axis=1, keepdims=True)
        oh = jnp.dot(p, xp_ref[0, :, h * C_:(h + 1) * C_],
                     preferred_element_type=jnp.float32)
        outs.append(oh * inv)
    return outs


def _concat_out(mask_ref, st_ref, xp_ref, *, bn, n):
    return jnp.concatenate(_attn_heads(mask_ref, st_ref, xp_ref, bn=bn, n=n),
                           axis=1)


def _mean_out(mask_ref, st_ref, xp_ref, *, bn, n):
    outs = _attn_heads(mask_ref, st_ref, xp_ref, bn=bn, n=n)
    acc = outs[0] + outs[1] + outs[2] + outs[3]
    return jnp.maximum(acc * (1.0 / H_), 0.0)


def _attn_cat_proj_body(mask_ref, st_ref, xp_ref, w_ref, a_ref,
                        xpo_ref, sto_ref, *, bn, n):
    out = _concat_out(mask_ref, st_ref, xp_ref, bn=bn, n=n)
    xpo = jnp.dot(out, w_ref[...], preferred_element_type=jnp.float32)
    xpo_ref[0] = xpo
    sto_ref[0] = _stj(a_ref[...], xpo)


def _attn_mean_body(mask_ref, st_ref, xp_ref, o_ref, *, bn, n):
    o_ref[0] = _mean_out(mask_ref, st_ref, xp_ref, bn=bn, n=n)


def _attn_mean_pool_body(mask_ref, st_ref, xp_ref, pk_ref, o_ref, g_ref, y_ref,
                         *, bn, n):
    j = pl.program_id(1)
    out = _mean_out(mask_ref, st_ref, xp_ref, bn=bn, n=n)
    o_ref[0] = out
    pk = pk_ref[0]
    kn = pk / (jnp.sqrt(jnp.sum(pk * pk)) + 1e-12)
    y = jnp.sum(out * kn[None, :], axis=1)
    y_ref[0, 0, pl.ds(j * bn, bn)] = y
    g_ref[0] = out * jnp.tanh(y)[:, None]


def _attn_mean_add_proj2_body(mask_ref, st_ref, xp_ref, d_ref, x2_ref,
                              w1_ref, w2_ref, a_ref, xpo_ref, sto_ref,
                              *, bn, n):
    hu = _mean_out(mask_ref, st_ref, xp_ref, bn=bn, n=n) + d_ref[0]
    xpo = jnp.dot(hu, w1_ref[...], preferred_element_type=jnp.float32)
    xpo = xpo + jnp.dot(x2_ref[0], w2_ref[...],
                        preferred_element_type=jnp.float32)
    xpo_ref[0] = xpo
    sto_ref[0] = _stj(a_ref[...], xpo)


def _attn_call(xp, st, mask8, mode, **kw):
    b, n, _ = xp.shape
    bn = 512
    grid = (b, n // bn)
    in_specs = [
        pl.BlockSpec((1, bn, n), lambda i, j: (i, j, 0)),
        pl.BlockSpec((1, 8, n), lambda i, j: (i, 0, 0)),
        pl.BlockSpec((1, n, HC_), lambda i, j: (i, 0, 0)),
    ]
    args = [mask8, st, xp]
    xpo_spec = pl.BlockSpec((1, bn, HC_), lambda i, j: (i, j, 0))
    sto_spec = pl.BlockSpec((1, 8, bn), lambda i, j: (i, 0, j))
    xpo_shapes = [jax.ShapeDtypeStruct((b, n, HC_), jnp.float32),
                  jax.ShapeDtypeStruct((b, 8, n), jnp.float32)]
    if mode == "cat_proj":
        body = functools.partial(_attn_cat_proj_body, bn=bn, n=n)
        in_specs += [pl.BlockSpec((HC_, HC_), lambda i, j: (0, 0)),
                     pl.BlockSpec((HC_, 8), lambda i, j: (0, 0))]
        args += [kw["w"], kw["astack"]]
        out_specs = [xpo_spec, sto_spec]
        out_shape = xpo_shapes
    elif mode == "mean":
        body = functools.partial(_attn_mean_body, bn=bn, n=n)
        out_specs = pl.BlockSpec((1, bn, C_), lambda i, j: (i, j, 0))
        out_shape = jax.ShapeDtypeStruct((b, n, C_), jnp.float32)
    elif mode == "mean_pool":
        body = functools.partial(_attn_mean_pool_body, bn=bn, n=n)
        in_specs.append(pl.BlockSpec((1, C_), lambda i, j: (0, 0)))
        args.append(kw["pk"])
        out_specs = [
            pl.BlockSpec((1, bn, C_), lambda i, j: (i, j, 0)),
            pl.BlockSpec((1, bn, C_), lambda i, j: (i, j, 0)),
            pl.BlockSpec((1, 1, n), lambda i, j: (i, 0, 0)),
        ]
        out_shape = [
            jax.ShapeDtypeStruct((b, n, C_), jnp.float32),
            jax.ShapeDtypeStruct((b, n, C_), jnp.float32),
            jax.ShapeDtypeStruct((b, 1, n), jnp.float32),
        ]
    elif mode == "mean_add_proj2":
        body = functools.partial(_attn_mean_add_proj2_body, bn=bn, n=n)
        in_specs += [
            pl.BlockSpec((1, bn, C_), lambda i, j: (i, j, 0)),
            pl.BlockSpec((1, bn, F_), lambda i, j: (i, j, 0)),
            pl.BlockSpec((C_, HC_), lambda i, j: (0, 0)),
            pl.BlockSpec((F_, HC_), lambda i, j: (0, 0)),
            pl.BlockSpec((HC_, 8), lambda i, j: (0, 0)),
        ]
        args += [kw["down"], kw["x2"], kw["w1"], kw["w2"], kw["astack"]]
        out_specs = [xpo_spec, sto_spec]
        out_shape = xpo_shapes
    return pl.pallas_call(
        body, grid=grid, in_specs=in_specs, out_specs=out_specs,
        out_shape=out_shape,
    )(*args)


_NC, _NS = 2, 16


def _sc_pool_gather(A2, hg2, idx2):
    mesh = plsc.VectorSubcoreMesh(core_axis_name="c", subcore_axis_name="s")

    @functools.partial(
        pl.kernel, mesh=mesh,
        out_type=[jax.ShapeDtypeStruct((B_ * K_, N_), jnp.float32),
                  jax.ShapeDtypeStruct((B_ * K_, C_), jnp.float32)],
        scratch_types=[
            pltpu.VMEM((64,), jnp.int32),
            pltpu.VMEM((64, N_), jnp.float32),
            pltpu.VMEM((64, C_), jnp.float32),
            pltpu.SemaphoreType.DMA,
            pltpu.SemaphoreType.DMA,
        ],
    )
    def k(a_hbm, hg_hbm, idx_hbm, ar_hbm, hp_hbm,
          rowabs_v, arows_v, hrows_v, sem1, sem2):
        w = lax.axis_index("s") * _NC + lax.axis_index("c")
        b = w // 8
        pltpu.sync_copy(idx_hbm.at[pl.ds(w * 64, 64)], rowabs_v)
        for q in range(4):
            sl = pl.ds(q * 16, 16)
            rowabs_v[sl] = rowabs_v[sl] + b * N_
        cp1 = pltpu.async_copy(a_hbm.at[rowabs_v], arows_v, sem1)
        cp2 = pltpu.async_copy(hg_hbm.at[rowabs_v], hrows_v, sem2)
        cp1.wait()
        cp2.wait()
        pltpu.sync_copy(arows_v, ar_hbm.at[pl.ds(w * 64, 64)])
        pltpu.sync_copy(hrows_v, hp_hbm.at[pl.ds(w * 64, 64)])

    return k(A2, hg2, idx2)


def _sc_scatter(src, idx2, zrows, cols, share_src):
    mesh = plsc.VectorSubcoreMesh(core_axis_name="c", subcore_axis_name="s")

    @functools.partial(
        pl.kernel, mesh=mesh,
        out_type=jax.ShapeDtypeStruct((B_ * N_, cols), jnp.float32),
        scratch_types=[
            pltpu.VMEM((64, cols), jnp.float32),
            pltpu.VMEM((32, cols), jnp.float32),
            pltpu.VMEM((32,), jnp.int32),
            pltpu.SemaphoreType.DMA,
        ],
    )
    def k(h_hbm, idx_hbm, z_hbm, out_hbm, zbuf, rbuf, iabs, sem):
        c = lax.axis_index("c")
        sid = lax.axis_index("s")
        pltpu.sync_copy(z_hbm, zbuf)
        for bb in range(2):
            b = c * 2 + bb
            pltpu.sync_copy(zbuf, out_hbm.at[pl.ds(b * N_ + sid * 64, 64)])
        plsc.subcore_barrier()
        for bb in range(2):
            b = c * 2 + bb
            base = b * K_ + sid * 32
            pltpu.sync_copy(idx_hbm.at[pl.ds(base, 32)], iabs)
            for q in range(2):
                sl = pl.ds(q * 16, 16)
                iabs[sl] = iabs[sl] + b * N_
            src_base = sid * 32 if share_src else base
            pltpu.sync_copy(h_hbm.at[pl.ds(src_base, 32)], rbuf)
            pltpu.async_copy(rbuf, out_hbm.at[iabs], sem).wait()

    return k(src, idx2, zrows)


def kernel(X, A, W0a, a0a_s, a0a_n, W0b, a0b_s, a0b_n, pk,
           W1a, a1a_s, a1a_n, W1b, a1b_s, a1b_n,
           Wua, aua_s, aua_n, Wub, aub_s, aub_n,
           Wea, aea_s, aea_n, Web, aeb_s, aeb_n):
    def wf(w):
        return w.reshape(w.shape[0], HC_)

    def av(a_s, a_n):
        z = jnp.zeros((HC_, 2 * H_), jnp.float32)
        for h in range(H_):
            z = z.at[h * C_:(h + 1) * C_, h].set(a_s[h])
            z = z.at[h * C_:(h + 1) * C_, H_ + h].set(a_n[h])
        return z

    xp, st, mask8 = _proj_call(X, wf(W0a), av(a0a_s, a0a_n), adj=A)
    xp, st = _attn_call(xp, st, mask8, "cat_proj",
                        w=wf(W0b), astack=av(a0b_s, a0b_n))
    down, hg, y = _attn_call(xp, st, mask8, "mean_pool", pk=pk.reshape(1, F_))

    _, idx = jax.lax.top_k(y[:, 0, :], K_)

    idxf = idx.reshape(B_ * K_)
    Ar2, hp2 = _sc_pool_gather(A.reshape(B_ * N_, N_),
                               hg.reshape(B_ * N_, C_), idxf)
    Ar = Ar2.reshape(B_, K_, N_)
    hp = hp2.reshape(B_, K_, C_)
    oh = _sc_scatter(jnp.eye(K_, dtype=jnp.float32), idxf,
                     jnp.zeros((64, K_), jnp.float32), K_,
                     True).reshape(B_, N_, K_)

    xp, st, mp8 = _proj_colsel_call(hp, Ar, oh, wf(W1a), av(a1a_s, a1a_n))
    xp, st = _attn_call(xp, st, mp8, "cat_proj",
                        w=wf(W1b), astack=av(a1b_s, a1b_n))
    h1b = _attn_call(xp, st, mp8, "mean")

    hu0 = _sc_scatter(h1b.reshape(B_ * K_, C_), idxf,
                      jnp.zeros((64, C_), jnp.float32), C_,
                      False).reshape(B_, N_, C_)

    xp, st = _proj_call(hu0, wf(Wua), av(aua_s, aua_n))
    xp, st = _attn_call(xp, st, mask8, "cat_proj",
                        w=wf(Wub), astack=av(aub_s, aub_n))
    xp, st = _attn_call(xp, st, mask8, "mean_add_proj2", down=down, x2=X,
                        w1=wf(Wea[:C_]), w2=wf(Wea[C_:]),
                        astack=av(aea_s, aea_n))
    xp, st = _attn_call(xp, st, mask8, "cat_proj",
                        w=wf(Web), astack=av(aeb_s, aeb_n))
    out = _attn_call(xp, st, mask8, "mean")
    return out
